# baseline (speedup 1.0000x reference)
import functools

import jax
import jax.numpy as jnp
from jax import lax
from jax.experimental import pallas as pl
from jax.experimental.pallas import tpu as pltpu
from jax.experimental.pallas import tpu_sc as plsc

_DIM = 64
_CTX = 5
_LANES = 16


def _sc_word2vec(B, ctx, dim):
    NW = 32
    b_per_w = B // NW
    CHUNK = 128
    n_chunks = b_per_w // CHUNK
    CB = CHUNK * ctx

    mesh = plsc.VectorSubcoreMesh(core_axis_name="c", subcore_axis_name="s")

    @functools.partial(
        pl.kernel,
        out_type=jax.ShapeDtypeStruct((B * ctx,), jnp.float32),
        mesh=mesh,
        scratch_types=[
            pltpu.VMEM((CHUNK,), jnp.int32),
            pltpu.VMEM((CB,), jnp.int32),
            pltpu.VMEM((CHUNK, dim), jnp.bfloat16),
            pltpu.VMEM((CB, dim), jnp.bfloat16),
            pltpu.VMEM((CB,), jnp.float32),
            pltpu.SemaphoreType.DMA,
            pltpu.SemaphoreType.DMA,
        ],
        compiler_params=pltpu.CompilerParams(
            needs_layout_passes=False, use_tc_tiling_on_sc=False),
    )
    def k(tgt_hbm, ctxi_hbm, ttab_hbm, ctab_hbm, out_hbm,
          idx_t, idx_c, trows, crows, outv, sem_t, sem_c):
        wid = lax.axis_index("s") * 2 + lax.axis_index("c")
        base_b = wid * b_per_w
        lane = lax.iota(jnp.int32, _LANES)

        for ci in range(n_chunks):
            off_b = pl.multiple_of(base_b + ci * CHUNK, CHUNK)
            off_c = pl.multiple_of((base_b + ci * CHUNK) * ctx, CB)
            pltpu.sync_copy(tgt_hbm.at[pl.ds(off_b, CHUNK)], idx_t)
            pltpu.sync_copy(ctxi_hbm.at[pl.ds(off_c, CB)], idx_c)
            cp_t = pltpu.async_copy(ttab_hbm.at[idx_t], trows, sem_t)
            cps = []
            for j in range(ctx):
                cps.append(pltpu.async_copy(
                    ctab_hbm.at[idx_c.at[pl.ds(j * CHUNK, CHUNK)]],
                    crows.at[pl.ds(j * CHUNK, CHUNK), :], sem_c))
            cp_t.wait()
            for cp in cps:
                cp.wait()

            lane_masks = [lane == j for j in range(_LANES)]

            def q_body(q, carry):
                res = jnp.zeros((_LANES,), jnp.float32)
                for j in range(_LANES):
                    p = q * _LANES + j
                    b = p // ctx
                    c = p - b * ctx
                    s = jnp.zeros((_LANES,), jnp.float32)
                    for k in range(dim // (2 * _LANES)):
                        sl = pl.ds(k * 2 * _LANES, 2 * _LANES)
                        w0, w1 = plsc.unpack(trows[b, sl], format=plsc.PackFormat.INTERLEAVED)
                        x0, x1 = plsc.unpack(crows[b * ctx + c, sl], format=plsc.PackFormat.INTERLEAVED)
                        s = s + w0 * x0 + w1 * x1
                    res = jnp.where(lane_masks[j], jnp.sum(s), res)
                outv[pl.ds(q * _LANES, _LANES)] = res
                return carry

            lax.fori_loop(0, CB // _LANES, q_body, 0)
            pltpu.sync_copy(outv, out_hbm.at[pl.ds(off_c, CB)])

    return k


def kernel(target, context, target_table, context_table):
    B, ctx = context.shape
    dim = target_table.shape[1]
    tgt = target.reshape(B).astype(jnp.int32)
    ctxi = context.reshape(B * ctx).astype(jnp.int32)
    out = _sc_word2vec(B, ctx, dim)(
        tgt, ctxi, target_table.astype(jnp.bfloat16),
        context_table.astype(jnp.bfloat16))
    return out.reshape(B, ctx)


# speedup vs baseline: 1.4757x; 1.4757x over previous
import functools

import jax
import jax.numpy as jnp
from jax import lax
from jax.experimental import pallas as pl
from jax.experimental.pallas import tpu as pltpu
from jax.experimental.pallas import tpu_sc as plsc

_LANES = 16
_NB = 2048
_NBH = _NB // 2
_NB_SH = 11
_NBH_SH = 10


def _sc_word2vec(B, ctx, dim):
    NW = 32
    b_per_w = B // NW
    CHUNK = 128
    n_chunks = b_per_w // CHUNK
    CB = CHUNK * ctx
    PD = 2 * dim

    mesh = plsc.VectorSubcoreMesh(core_axis_name="c", subcore_axis_name="s")

    @functools.partial(
        pl.kernel,
        out_type=jax.ShapeDtypeStruct((B * ctx,), jnp.float32),
        mesh=mesh,
        scratch_types=[
            pltpu.VMEM((CHUNK,), jnp.int32),
            pltpu.VMEM((CB,), jnp.int32),
            pltpu.VMEM((CHUNK,), jnp.int32),
            pltpu.VMEM((CHUNK, PD), jnp.float32),
            pltpu.VMEM((CB, PD), jnp.float32),
            pltpu.VMEM((CB,), jnp.float32),
            pltpu.SemaphoreType.DMA,
            pltpu.SemaphoreType.DMA,
        ],
        compiler_params=pltpu.CompilerParams(needs_layout_passes=False),
    )
    def k(tgt_hbm, ctxi_hbm, ttab_hbm, ctab_hbm, out_hbm,
          idx_t, idx_c, idx_t2, trows, crows, outv, sem_t, sem_c):
        wid = lax.axis_index("s") * 2 + lax.axis_index("c")
        base_b = wid * b_per_w
        lane = lax.iota(jnp.int32, _LANES)

        for ci in range(n_chunks):
            off_b = pl.multiple_of(base_b + ci * CHUNK, CHUNK)
            off_c = pl.multiple_of((base_b + ci * CHUNK) * ctx, CB)
            pltpu.sync_copy(tgt_hbm.at[pl.ds(off_b, CHUNK)], idx_t)
            pltpu.sync_copy(ctxi_hbm.at[pl.ds(off_c, CB)], idx_c)
            def _srow(v):
                return (lax.shift_left(
                    lax.shift_right_logical(v, _NB_SH), _NBH_SH)
                        + (v & (_NBH - 1)))
            def halve_t(i, carry):
                v = idx_t[pl.ds(i * _LANES, _LANES)]
                idx_t2[pl.ds(i * _LANES, _LANES)] = _srow(v)
                return carry
            lax.fori_loop(0, CHUNK // _LANES, halve_t, 0)
            cp_t = pltpu.async_copy(ttab_hbm.at[idx_t2], trows, sem_t)
            cps = []
            for j in range(ctx):
                cps.append(pltpu.async_copy(
                    ctab_hbm.at[idx_c.at[pl.ds(j * CHUNK, CHUNK)]],
                    crows.at[pl.ds(j * CHUNK, CHUNK), :], sem_c))
            cp_t.wait()
            for cp in cps:
                cp.wait()

            def g_body(g, carry):
                b_vec = g * _LANES + lane
                tv = idx_t[pl.ds(g * _LANES, _LANES)]
                tpar = (lax.shift_right_logical(tv, _NBH_SH) & 1) * dim
                crow = [b_vec * ctx + c for c in range(ctx)]
                accs = [jnp.zeros((_LANES,), jnp.float32) for _ in range(ctx)]
                for e in range(dim):
                    e_vec = jnp.full((_LANES,), e, jnp.int32)
                    w = plsc.load_gather(trows, [b_vec, tpar + e])
                    for c in range(ctx):
                        x = plsc.load_gather(crows, [crow[c], e_vec])
                        accs[c] = accs[c] + w * x
                for c in range(ctx):
                    plsc.store_scatter(outv, [crow[c]], accs[c])
                return carry

            lax.fori_loop(0, CHUNK // _LANES, g_body, 0)
            pltpu.sync_copy(outv, out_hbm.at[pl.ds(off_c, CB)])

    return k


def _tc_repack(V, dim):
    NB = _NB
    grid = (V + NB - 1) // NB

    def body(in_ref, out_ref):
        x = in_ref[...]
        ident = jnp.eye(dim, dtype=jnp.float32)
        xt = lax.dot_general(x, ident, (((0,), (0,)), ((), ())),
                             preferred_element_type=jnp.float32)
        out_ref[:, 0:dim] = xt[0:NB // 2]
        out_ref[:, dim:2 * dim] = xt[NB // 2:NB]

    return pl.pallas_call(
        body,
        grid=(grid,),
        in_specs=[pl.BlockSpec((dim, NB), lambda i: (0, i))],
        out_specs=pl.BlockSpec((NB // 2, 2 * dim), lambda i: (i, 0)),
        out_shape=jax.ShapeDtypeStruct((grid * (NB // 2), 2 * dim),
                                       jnp.float32),
    )


def kernel(target, context, target_table, context_table):
    B, ctx = context.shape
    V, dim = target_table.shape
    tgt = target.reshape(B).astype(jnp.int32)
    ctxi = context.reshape(B * ctx).astype(jnp.int32)
    ttab2 = _tc_repack(V, dim)(target_table.T)
    ctab2 = jnp.pad(context_table, ((0, 0), (0, 2 * dim - dim)))
    out = _sc_word2vec(B, ctx, dim)(tgt, ctxi, ttab2, ctab2)
    return out.reshape(B, ctx)


# speedup vs baseline: 1.6840x; 1.1411x over previous
import functools

import jax
import jax.numpy as jnp
from jax import lax
from jax.experimental import pallas as pl
from jax.experimental.pallas import tpu as pltpu
from jax.experimental.pallas import tpu_sc as plsc

_LANES = 16
_NB = 4096
_NBH = _NB // 2
_NB_SH = 12
_NBH_SH = 11


def _sc_word2vec(B, ctx, dim):
    NW = 32
    b_per_w = B // NW
    CHUNK = 128
    n_chunks = b_per_w // CHUNK
    CB = CHUNK * ctx
    PD = 2 * dim

    mesh = plsc.VectorSubcoreMesh(core_axis_name="c", subcore_axis_name="s")

    @functools.partial(
        pl.kernel,
        out_type=jax.ShapeDtypeStruct((B * ctx,), jnp.float32),
        mesh=mesh,
        scratch_types=[
            pltpu.VMEM((CHUNK,), jnp.int32),
            pltpu.VMEM((CB,), jnp.int32),
            pltpu.VMEM((CHUNK,), jnp.int32),
            pltpu.VMEM((CHUNK, PD), jnp.float32),
            pltpu.VMEM((CB, PD), jnp.float32),
            pltpu.VMEM((CB,), jnp.float32),
            pltpu.SemaphoreType.DMA,
            pltpu.SemaphoreType.DMA,
        ],
        compiler_params=pltpu.CompilerParams(needs_layout_passes=False),
    )
    def k(tgt_hbm, ctxi_hbm, ttab_hbm, ctab_hbm, out_hbm,
          idx_t, idx_c, idx_t2, trows, crows, outv, sem_t, sem_c):
        wid = lax.axis_index("s") * 2 + lax.axis_index("c")
        base_b = wid * b_per_w
        lane = lax.iota(jnp.int32, _LANES)

        for ci in range(n_chunks):
            off_b = pl.multiple_of(base_b + ci * CHUNK, CHUNK)
            off_c = pl.multiple_of((base_b + ci * CHUNK) * ctx, CB)
            pltpu.sync_copy(tgt_hbm.at[pl.ds(off_b, CHUNK)], idx_t)
            pltpu.sync_copy(ctxi_hbm.at[pl.ds(off_c, CB)], idx_c)
            def _srow(v):
                return (lax.shift_left(
                    lax.shift_right_logical(v, _NB_SH), _NBH_SH)
                        + (v & (_NBH - 1)))
            def halve_t(i, carry):
                v = idx_t[pl.ds(i * _LANES, _LANES)]
                idx_t2[pl.ds(i * _LANES, _LANES)] = _srow(v)
                return carry
            lax.fori_loop(0, CHUNK // _LANES, halve_t, 0)
            cp_t = pltpu.async_copy(ttab_hbm.at[idx_t2], trows, sem_t)
            cps = []
            for j in range(ctx):
                cps.append(pltpu.async_copy(
                    ctab_hbm.at[idx_c.at[pl.ds(j * CHUNK, CHUNK)]],
                    crows.at[pl.ds(j * CHUNK, CHUNK), :], sem_c))
            cp_t.wait()
            for cp in cps:
                cp.wait()

            def g_body(g, carry):
                b_vec = g * _LANES + lane
                tv = idx_t[pl.ds(g * _LANES, _LANES)]
                tpar = (lax.shift_right_logical(tv, _NBH_SH) & 1) * dim
                crow = [b_vec * ctx + c for c in range(ctx)]
                accs = [jnp.zeros((_LANES,), jnp.float32) for _ in range(ctx)]
                for e in range(dim):
                    e_vec = jnp.full((_LANES,), e, jnp.int32)
                    w = plsc.load_gather(trows, [b_vec, tpar + e])
                    for c in range(ctx):
                        x = plsc.load_gather(crows, [crow[c], e_vec])
                        accs[c] = accs[c] + w * x
                for c in range(ctx):
                    plsc.store_scatter(outv, [crow[c]], accs[c])
                return carry

            lax.fori_loop(0, CHUNK // _LANES, g_body, 0)
            pltpu.sync_copy(outv, out_hbm.at[pl.ds(off_c, CB)])

    return k


def _tc_repack(V, dim):
    NB = _NB
    grid = (V + NB - 1) // NB

    def body(in_ref, out_ref):
        x = in_ref[...]
        ident = jnp.eye(dim, dtype=jnp.float32)
        xt = lax.dot_general(x, ident, (((0,), (0,)), ((), ())),
                             preferred_element_type=jnp.float32)
        out_ref[:, 0:dim] = xt[0:NB // 2]
        out_ref[:, dim:2 * dim] = xt[NB // 2:NB]

    return pl.pallas_call(
        body,
        grid=(grid,),
        in_specs=[pl.BlockSpec((dim, NB), lambda i: (0, i))],
        out_specs=pl.BlockSpec((NB // 2, 2 * dim), lambda i: (i, 0)),
        out_shape=jax.ShapeDtypeStruct((grid * (NB // 2), 2 * dim),
                                       jnp.float32),
    )


def kernel(target, context, target_table, context_table):
    B, ctx = context.shape
    V, dim = target_table.shape
    tgt = target.reshape(B).astype(jnp.int32)
    ctxi = context.reshape(B * ctx).astype(jnp.int32)
    ttab2 = _tc_repack(V, dim)(target_table.T)
    ctab2 = jnp.pad(context_table, ((0, 0), (0, 2 * dim - dim)))
    out = _sc_word2vec(B, ctx, dim)(tgt, ctxi, ttab2, ctab2)
    return out.reshape(B, ctx)


# speedup vs baseline: 1.8112x; 1.0755x over previous
import functools

import jax
import jax.numpy as jnp
from jax import lax
from jax.experimental import pallas as pl
from jax.experimental.pallas import tpu as pltpu
from jax.experimental.pallas import tpu_sc as plsc

_LANES = 16
_NB = 8192
_NBH = _NB // 2
_NB_SH = 13
_NBH_SH = 12


def _sc_word2vec(B, ctx, dim):
    NW = 32
    b_per_w = B // NW
    CHUNK = 128
    n_chunks = b_per_w // CHUNK
    CB = CHUNK * ctx
    PD = 2 * dim

    mesh = plsc.VectorSubcoreMesh(core_axis_name="c", subcore_axis_name="s")

    @functools.partial(
        pl.kernel,
        out_type=jax.ShapeDtypeStruct((B * ctx,), jnp.float32),
        mesh=mesh,
        scratch_types=[
            pltpu.VMEM((CHUNK,), jnp.int32),
            pltpu.VMEM((CB,), jnp.int32),
            pltpu.VMEM((CHUNK,), jnp.int32),
            pltpu.VMEM((CHUNK, PD), jnp.float32),
            pltpu.VMEM((CB, PD), jnp.float32),
            pltpu.VMEM((CB,), jnp.float32),
            pltpu.SemaphoreType.DMA,
            pltpu.SemaphoreType.DMA,
        ],
        compiler_params=pltpu.CompilerParams(needs_layout_passes=False),
    )
    def k(tgt_hbm, ctxi_hbm, ttab_hbm, ctab_hbm, out_hbm,
          idx_t, idx_c, idx_t2, trows, crows, outv, sem_t, sem_c):
        wid = lax.axis_index("s") * 2 + lax.axis_index("c")
        base_b = wid * b_per_w
        lane = lax.iota(jnp.int32, _LANES)

        for ci in range(n_chunks):
            off_b = pl.multiple_of(base_b + ci * CHUNK, CHUNK)
            off_c = pl.multiple_of((base_b + ci * CHUNK) * ctx, CB)
            pltpu.sync_copy(tgt_hbm.at[pl.ds(off_b, CHUNK)], idx_t)
            pltpu.sync_copy(ctxi_hbm.at[pl.ds(off_c, CB)], idx_c)
            def _srow(v):
                return (lax.shift_left(
                    lax.shift_right_logical(v, _NB_SH), _NBH_SH)
                        + (v & (_NBH - 1)))
            def halve_t(i, carry):
                v = idx_t[pl.ds(i * _LANES, _LANES)]
                idx_t2[pl.ds(i * _LANES, _LANES)] = _srow(v)
                return carry
            lax.fori_loop(0, CHUNK // _LANES, halve_t, 0)
            cp_t = pltpu.async_copy(ttab_hbm.at[idx_t2], trows, sem_t)
            cps = []
            for j in range(ctx):
                cps.append(pltpu.async_copy(
                    ctab_hbm.at[idx_c.at[pl.ds(j * CHUNK, CHUNK)]],
                    crows.at[pl.ds(j * CHUNK, CHUNK), :], sem_c))
            cp_t.wait()
            for cp in cps:
                cp.wait()

            def g_body(g, carry):
                b_vec = g * _LANES + lane
                tv = idx_t[pl.ds(g * _LANES, _LANES)]
                tpar = (lax.shift_right_logical(tv, _NBH_SH) & 1) * dim
                crow = [b_vec * ctx + c for c in range(ctx)]
                accs = [jnp.zeros((_LANES,), jnp.float32) for _ in range(ctx)]
                for e in range(dim):
                    e_vec = jnp.full((_LANES,), e, jnp.int32)
                    w = plsc.load_gather(trows, [b_vec, tpar + e])
                    for c in range(ctx):
                        x = plsc.load_gather(crows, [crow[c], e_vec])
                        accs[c] = accs[c] + w * x
                for c in range(ctx):
                    plsc.store_scatter(outv, [crow[c]], accs[c])
                return carry

            lax.fori_loop(0, CHUNK // _LANES, g_body, 0)
            pltpu.sync_copy(outv, out_hbm.at[pl.ds(off_c, CB)])

    return k


def _tc_repack(V, dim):
    NB = _NB
    grid = (V + NB - 1) // NB

    def body(in_ref, out_ref):
        x = in_ref[...]
        ident = jnp.eye(dim, dtype=jnp.float32)
        xt = lax.dot_general(x, ident, (((0,), (0,)), ((), ())),
                             preferred_element_type=jnp.float32)
        out_ref[:, 0:dim] = xt[0:NB // 2]
        out_ref[:, dim:2 * dim] = xt[NB // 2:NB]

    return pl.pallas_call(
        body,
        grid=(grid,),
        in_specs=[pl.BlockSpec((dim, NB), lambda i: (0, i))],
        out_specs=pl.BlockSpec((NB // 2, 2 * dim), lambda i: (i, 0)),
        out_shape=jax.ShapeDtypeStruct((grid * (NB // 2), 2 * dim),
                                       jnp.float32),
    )


def kernel(target, context, target_table, context_table):
    B, ctx = context.shape
    V, dim = target_table.shape
    tgt = target.reshape(B).astype(jnp.int32)
    ctxi = context.reshape(B * ctx).astype(jnp.int32)
    ttab2 = _tc_repack(V, dim)(target_table.T)
    ctab2 = jnp.pad(context_table, ((0, 0), (0, 2 * dim - dim)))
    out = _sc_word2vec(B, ctx, dim)(tgt, ctxi, ttab2, ctab2)
    return out.reshape(B, ctx)


# speedup vs baseline: 2.1861x; 1.2070x over previous
import functools

import jax
import jax.numpy as jnp
from jax import lax
from jax.experimental import pallas as pl
from jax.experimental.pallas import tpu as pltpu
from jax.experimental.pallas import tpu_sc as plsc

_LANES = 16
_NB = 8192
_NBH = _NB // 2
_NB_SH = 13
_NBH_SH = 12


def _sc_word2vec(B, ctx, dim):
    NW = 32
    b_per_w = B // NW
    CHUNK = 128
    n_chunks = b_per_w // CHUNK
    CB = CHUNK * ctx
    PD = 2 * dim

    mesh = plsc.VectorSubcoreMesh(core_axis_name="c", subcore_axis_name="s")

    @functools.partial(
        pl.kernel,
        out_type=jax.ShapeDtypeStruct((B * ctx,), jnp.float32),
        mesh=mesh,
        scratch_types=[
            pltpu.VMEM((CHUNK,), jnp.int32),
            pltpu.VMEM((CB,), jnp.int32),
            pltpu.VMEM((CHUNK,), jnp.int32),
            pltpu.VMEM((CHUNK, PD), jnp.float32),
            pltpu.VMEM((CB, PD), jnp.float32),
            pltpu.VMEM((CB,), jnp.float32),
            pltpu.SemaphoreType.DMA,
            pltpu.SemaphoreType.DMA,
        ],
        compiler_params=pltpu.CompilerParams(needs_layout_passes=False),
    )
    def k(tgt_hbm, ctxi_hbm, ttab_hbm, ctab_hbm, out_hbm,
          idx_t, idx_c, idx_t2, trows, crows, outv, sem_t, sem_c):
        wid = lax.axis_index("s") * 2 + lax.axis_index("c")
        base_b = wid * b_per_w
        lane = lax.iota(jnp.int32, _LANES)

        for ci in range(n_chunks):
            off_b = pl.multiple_of(base_b + ci * CHUNK, CHUNK)
            off_c = pl.multiple_of((base_b + ci * CHUNK) * ctx, CB)
            pltpu.sync_copy(tgt_hbm.at[pl.ds(off_b, CHUNK)], idx_t)
            pltpu.sync_copy(ctxi_hbm.at[pl.ds(off_c, CB)], idx_c)
            def _srow(v):
                return (lax.shift_left(
                    lax.shift_right_logical(v, _NB_SH), _NBH_SH)
                        + (v & (_NBH - 1)))
            def halve_t(i, carry):
                v = idx_t[pl.ds(i * _LANES, _LANES)]
                idx_t2[pl.ds(i * _LANES, _LANES)] = _srow(v)
                return carry
            lax.fori_loop(0, CHUNK // _LANES, halve_t, 0)
            cp_t = pltpu.async_copy(ttab_hbm.at[idx_t2], trows, sem_t)
            cps = []
            for j in range(ctx):
                cps.append(pltpu.async_copy(
                    ctab_hbm.at[idx_c.at[pl.ds(j * CHUNK, CHUNK)]],
                    crows.at[pl.ds(j * CHUNK, CHUNK), :], sem_c))
            cp_t.wait()
            for cp in cps:
                cp.wait()

            def g_body(g, carry):
                b_vec = g * _LANES + lane
                tv = idx_t[pl.ds(g * _LANES, _LANES)]
                tpar = (lax.shift_right_logical(tv, _NBH_SH) & 1) * dim
                crow = [b_vec * ctx + c for c in range(ctx)]
                accs = [jnp.zeros((_LANES,), jnp.float32) for _ in range(ctx)]
                for e in range(dim):
                    e_vec = jnp.full((_LANES,), e, jnp.int32)
                    w = plsc.load_gather(trows, [b_vec, tpar + e])
                    for c in range(ctx):
                        x = plsc.load_gather(crows, [crow[c], e_vec])
                        accs[c] = accs[c] + w * x
                for c in range(ctx):
                    plsc.store_scatter(outv, [crow[c]], accs[c])
                return carry

            lax.fori_loop(0, CHUNK // _LANES, g_body, 0)
            pltpu.sync_copy(outv, out_hbm.at[pl.ds(off_c, CB)])

    return k


def _tc_repack(V, dim):
    NB = _NB
    grid = (V + NB - 1) // NB

    def body(in_ref, out_ref):
        x = in_ref[...]
        ident = jnp.eye(dim, dtype=jnp.float32)
        xt = lax.dot_general(x, ident, (((0,), (0,)), ((), ())),
                             preferred_element_type=jnp.float32)
        out_ref[:, 0:dim] = xt[0:NB // 2]
        out_ref[:, dim:2 * dim] = xt[NB // 2:NB]

    return pl.pallas_call(
        body,
        grid=(grid,),
        in_specs=[pl.BlockSpec((dim, NB), lambda i: (0, i))],
        out_specs=pl.BlockSpec((NB // 2, 2 * dim), lambda i: (i, 0)),
        out_shape=jax.ShapeDtypeStruct((grid * (NB // 2), 2 * dim),
                                       jnp.float32),
    )


def kernel(target, context, target_table, context_table):
    B, ctx = context.shape
    V, dim = target_table.shape
    tgt = target.reshape(B).astype(jnp.int32)
    ctxi = context.reshape(B * ctx).astype(jnp.int32)
    repack = _tc_repack(V, dim)
    ttab2 = repack(target_table.T)
    ctab2 = repack(context_table.T)
    out = _sc_word2vec(B, ctx, dim)(tgt, ctxi, ttab2, ctab2)
    return out.reshape(B, ctx)


# speedup vs baseline: 2.1923x; 1.0028x over previous
import functools

import jax
import jax.numpy as jnp
from jax import lax
from jax.experimental import pallas as pl
from jax.experimental.pallas import tpu as pltpu
from jax.experimental.pallas import tpu_sc as plsc

_LANES = 16
_NB = 8192
_NBH = _NB // 2
_NB_SH = 13
_NBH_SH = 12


def _sc_word2vec(B, ctx, dim):
    NW = 32
    b_per_w = B // NW
    CHUNK = 128
    n_chunks = b_per_w // CHUNK
    CB = CHUNK * ctx
    PD = 2 * dim

    mesh = plsc.VectorSubcoreMesh(core_axis_name="c", subcore_axis_name="s")

    @functools.partial(
        pl.kernel,
        out_type=jax.ShapeDtypeStruct((B * ctx,), jnp.float32),
        mesh=mesh,
        scratch_types=[
            pltpu.VMEM((CHUNK,), jnp.int32),
            pltpu.VMEM((CB,), jnp.int32),
            pltpu.VMEM((CHUNK,), jnp.int32),
            pltpu.VMEM((CB,), jnp.int32),
            pltpu.VMEM((CHUNK, PD), jnp.float32),
            pltpu.VMEM((CB, PD), jnp.float32),
            pltpu.VMEM((CB,), jnp.float32),
            pltpu.SemaphoreType.DMA,
            pltpu.SemaphoreType.DMA,
        ],
        compiler_params=pltpu.CompilerParams(needs_layout_passes=False),
    )
    def k(tgt_hbm, ctxi_hbm, ttab_hbm, ctab_hbm, out_hbm,
          idx_t, idx_c, idx_t2, idx_c2, trows, crows, outv, sem_t, sem_c):
        wid = lax.axis_index("s") * 2 + lax.axis_index("c")
        base_b = wid * b_per_w
        lane = lax.iota(jnp.int32, _LANES)

        for ci in range(n_chunks):
            off_b = pl.multiple_of(base_b + ci * CHUNK, CHUNK)
            off_c = pl.multiple_of((base_b + ci * CHUNK) * ctx, CB)
            pltpu.sync_copy(tgt_hbm.at[pl.ds(off_b, CHUNK)], idx_t)
            pltpu.sync_copy(ctxi_hbm.at[pl.ds(off_c, CB)], idx_c)
            def _srow(v):
                return (lax.shift_left(
                    lax.shift_right_logical(v, _NB_SH), _NBH_SH)
                        + (v & (_NBH - 1)))
            def halve_t(i, carry):
                v = idx_t[pl.ds(i * _LANES, _LANES)]
                idx_t2[pl.ds(i * _LANES, _LANES)] = _srow(v)
                return carry
            def halve_c(i, carry):
                v = idx_c[pl.ds(i * _LANES, _LANES)]
                idx_c2[pl.ds(i * _LANES, _LANES)] = _srow(v)
                return carry
            lax.fori_loop(0, CHUNK // _LANES, halve_t, 0)
            lax.fori_loop(0, CB // _LANES, halve_c, 0)
            cp_t = pltpu.async_copy(ttab_hbm.at[idx_t2], trows, sem_t)
            cps = []
            for j in range(ctx):
                cps.append(pltpu.async_copy(
                    ctab_hbm.at[idx_c2.at[pl.ds(j * CHUNK, CHUNK)]],
                    crows.at[pl.ds(j * CHUNK, CHUNK), :], sem_c))
            cp_t.wait()
            for cp in cps:
                cp.wait()

            def g_body(g, carry):
                b_vec = g * _LANES + lane
                tv = idx_t[pl.ds(g * _LANES, _LANES)]
                tpar = (lax.shift_right_logical(tv, _NBH_SH) & 1) * dim
                crow = [b_vec * ctx + c for c in range(ctx)]
                cpar = []
                for c in range(ctx):
                    cv = plsc.load_gather(idx_c, [crow[c]])
                    cpar.append(
                        (lax.shift_right_logical(cv, _NBH_SH) & 1) * dim)
                accs = [jnp.zeros((_LANES,), jnp.float32) for _ in range(ctx)]
                for e in range(dim):
                    w = plsc.load_gather(trows, [b_vec, tpar + e])
                    for c in range(ctx):
                        x = plsc.load_gather(crows, [crow[c], cpar[c] + e])
                        accs[c] = accs[c] + w * x
                for c in range(ctx):
                    plsc.store_scatter(outv, [crow[c]], accs[c])
                return carry

            lax.fori_loop(0, CHUNK // _LANES, g_body, 0)
            pltpu.sync_copy(outv, out_hbm.at[pl.ds(off_c, CB)])

    return k


def _tc_repack(V, dim):
    NB = _NB
    grid = (V + NB - 1) // NB

    def body(in_ref, out_ref):
        x = in_ref[...]
        ident = jnp.eye(dim, dtype=jnp.float32)
        xt = lax.dot_general(x, ident, (((0,), (0,)), ((), ())),
                             preferred_element_type=jnp.float32)
        out_ref[:, 0:dim] = xt[0:NB // 2]
        out_ref[:, dim:2 * dim] = xt[NB // 2:NB]

    return pl.pallas_call(
        body,
        grid=(grid,),
        in_specs=[pl.BlockSpec((dim, NB), lambda i: (0, i))],
        out_specs=pl.BlockSpec((NB // 2, 2 * dim), lambda i: (i, 0)),
        out_shape=jax.ShapeDtypeStruct((grid * (NB // 2), 2 * dim),
                                       jnp.float32),
    )


def kernel(target, context, target_table, context_table):
    B, ctx = context.shape
    V, dim = target_table.shape
    tgt = target.reshape(B).astype(jnp.int32)
    ctxi = context.reshape(B * ctx).astype(jnp.int32)
    repack = _tc_repack(V, dim)
    ttab2 = repack(target_table.T)
    ctab2 = repack(context_table.T)
    out = _sc_word2vec(B, ctx, dim)(tgt, ctxi, ttab2, ctab2)
    return out.reshape(B, ctx)


# speedup vs baseline: 2.4246x; 1.1060x over previous
import functools

import jax
import jax.numpy as jnp
from jax import lax
from jax.experimental import pallas as pl
from jax.experimental.pallas import tpu as pltpu
from jax.experimental.pallas import tpu_sc as plsc

_LANES = 16
_NB = 16384
_NBH = _NB // 2
_NB_SH = 14
_NBH_SH = 13


def _sc_word2vec(B, ctx, dim):
    NW = 32
    b_per_w = B // NW
    CHUNK = 128
    n_chunks = b_per_w // CHUNK
    CB = CHUNK * ctx
    PD = 2 * dim

    mesh = plsc.VectorSubcoreMesh(core_axis_name="c", subcore_axis_name="s")

    @functools.partial(
        pl.kernel,
        out_type=jax.ShapeDtypeStruct((B * ctx,), jnp.float32),
        mesh=mesh,
        scratch_types=[
            pltpu.VMEM((CHUNK,), jnp.int32),
            pltpu.VMEM((CB,), jnp.int32),
            pltpu.VMEM((CHUNK,), jnp.int32),
            pltpu.VMEM((CB,), jnp.int32),
            pltpu.VMEM((CHUNK, PD), jnp.float32),
            pltpu.VMEM((CB, PD), jnp.float32),
            pltpu.VMEM((CB,), jnp.float32),
            pltpu.SemaphoreType.DMA,
            pltpu.SemaphoreType.DMA,
        ],
        compiler_params=pltpu.CompilerParams(needs_layout_passes=False),
    )
    def k(tgt_hbm, ctxi_hbm, ttab_hbm, ctab_hbm, out_hbm,
          idx_t, idx_c, idx_t2, idx_c2, trows, crows, outv, sem_t, sem_c):
        wid = lax.axis_index("s") * 2 + lax.axis_index("c")
        base_b = wid * b_per_w
        lane = lax.iota(jnp.int32, _LANES)

        for ci in range(n_chunks):
            off_b = pl.multiple_of(base_b + ci * CHUNK, CHUNK)
            off_c = pl.multiple_of((base_b + ci * CHUNK) * ctx, CB)
            pltpu.sync_copy(tgt_hbm.at[pl.ds(off_b, CHUNK)], idx_t)
            pltpu.sync_copy(ctxi_hbm.at[pl.ds(off_c, CB)], idx_c)
            def _srow(v):
                return (lax.shift_left(
                    lax.shift_right_logical(v, _NB_SH), _NBH_SH)
                        + (v & (_NBH - 1)))
            def halve_t(i, carry):
                v = idx_t[pl.ds(i * _LANES, _LANES)]
                idx_t2[pl.ds(i * _LANES, _LANES)] = _srow(v)
                return carry
            def halve_c(i, carry):
                v = idx_c[pl.ds(i * _LANES, _LANES)]
                idx_c2[pl.ds(i * _LANES, _LANES)] = _srow(v)
                return carry
            lax.fori_loop(0, CHUNK // _LANES, halve_t, 0)
            lax.fori_loop(0, CB // _LANES, halve_c, 0)
            cp_t = pltpu.async_copy(ttab_hbm.at[idx_t2], trows, sem_t)
            cps = []
            for j in range(ctx):
                cps.append(pltpu.async_copy(
                    ctab_hbm.at[idx_c2.at[pl.ds(j * CHUNK, CHUNK)]],
                    crows.at[pl.ds(j * CHUNK, CHUNK), :], sem_c))
            cp_t.wait()
            for cp in cps:
                cp.wait()

            def g_body(g, carry):
                b_vec = g * _LANES + lane
                tv = idx_t[pl.ds(g * _LANES, _LANES)]
                tpar = (lax.shift_right_logical(tv, _NBH_SH) & 1) * dim
                crow = [b_vec * ctx + c for c in range(ctx)]
                cpar = []
                for c in range(ctx):
                    cv = plsc.load_gather(idx_c, [crow[c]])
                    cpar.append(
                        (lax.shift_right_logical(cv, _NBH_SH) & 1) * dim)
                accs = [jnp.zeros((_LANES,), jnp.float32) for _ in range(ctx)]
                for e in range(dim):
                    w = plsc.load_gather(trows, [b_vec, tpar + e])
                    for c in range(ctx):
                        x = plsc.load_gather(crows, [crow[c], cpar[c] + e])
                        accs[c] = accs[c] + w * x
                for c in range(ctx):
                    plsc.store_scatter(outv, [crow[c]], accs[c])
                return carry

            lax.fori_loop(0, CHUNK // _LANES, g_body, 0)
            pltpu.sync_copy(outv, out_hbm.at[pl.ds(off_c, CB)])

    return k


def _tc_repack(V, dim):
    NB = _NB
    grid = (V + NB - 1) // NB

    def body(in_ref, out_ref):
        x = in_ref[...]
        ident = jnp.eye(dim, dtype=jnp.float32)
        xt = lax.dot_general(x, ident, (((0,), (0,)), ((), ())),
                             preferred_element_type=jnp.float32)
        out_ref[:, 0:dim] = xt[0:NB // 2]
        out_ref[:, dim:2 * dim] = xt[NB // 2:NB]

    return pl.pallas_call(
        body,
        grid=(grid,),
        in_specs=[pl.BlockSpec((dim, NB), lambda i: (0, i))],
        out_specs=pl.BlockSpec((NB // 2, 2 * dim), lambda i: (i, 0)),
        out_shape=jax.ShapeDtypeStruct((grid * (NB // 2), 2 * dim),
                                       jnp.float32),
    )


def kernel(target, context, target_table, context_table):
    B, ctx = context.shape
    V, dim = target_table.shape
    tgt = target.reshape(B).astype(jnp.int32)
    ctxi = context.reshape(B * ctx).astype(jnp.int32)
    repack = _tc_repack(V, dim)
    ttab2 = repack(target_table.T)
    ctab2 = repack(context_table.T)
    out = _sc_word2vec(B, ctx, dim)(tgt, ctxi, ttab2, ctab2)
    return out.reshape(B, ctx)


# speedup vs baseline: 2.5440x; 1.0492x over previous
import functools

import jax
import jax.numpy as jnp
from jax import lax
from jax.experimental import pallas as pl
from jax.experimental.pallas import tpu as pltpu
from jax.experimental.pallas import tpu_sc as plsc

_LANES = 16
_NB = 32768
_NBH = _NB // 2
_NB_SH = 15
_NBH_SH = 14


def _sc_word2vec(B, ctx, dim):
    NW = 32
    b_per_w = B // NW
    CHUNK = 128
    n_chunks = b_per_w // CHUNK
    CB = CHUNK * ctx
    PD = 2 * dim

    mesh = plsc.VectorSubcoreMesh(core_axis_name="c", subcore_axis_name="s")

    @functools.partial(
        pl.kernel,
        out_type=jax.ShapeDtypeStruct((B * ctx,), jnp.float32),
        mesh=mesh,
        scratch_types=[
            pltpu.VMEM((CHUNK,), jnp.int32),
            pltpu.VMEM((CB,), jnp.int32),
            pltpu.VMEM((CHUNK,), jnp.int32),
            pltpu.VMEM((CB,), jnp.int32),
            pltpu.VMEM((CHUNK, PD), jnp.float32),
            pltpu.VMEM((CB, PD), jnp.float32),
            pltpu.VMEM((CB,), jnp.float32),
            pltpu.SemaphoreType.DMA,
            pltpu.SemaphoreType.DMA,
        ],
        compiler_params=pltpu.CompilerParams(needs_layout_passes=False),
    )
    def k(tgt_hbm, ctxi_hbm, ttab_hbm, ctab_hbm, out_hbm,
          idx_t, idx_c, idx_t2, idx_c2, trows, crows, outv, sem_t, sem_c):
        wid = lax.axis_index("s") * 2 + lax.axis_index("c")
        base_b = wid * b_per_w
        lane = lax.iota(jnp.int32, _LANES)

        for ci in range(n_chunks):
            off_b = pl.multiple_of(base_b + ci * CHUNK, CHUNK)
            off_c = pl.multiple_of((base_b + ci * CHUNK) * ctx, CB)
            pltpu.sync_copy(tgt_hbm.at[pl.ds(off_b, CHUNK)], idx_t)
            pltpu.sync_copy(ctxi_hbm.at[pl.ds(off_c, CB)], idx_c)
            def _srow(v):
                return (lax.shift_left(
                    lax.shift_right_logical(v, _NB_SH), _NBH_SH)
                        + (v & (_NBH - 1)))
            def halve_t(i, carry):
                v = idx_t[pl.ds(i * _LANES, _LANES)]
                idx_t2[pl.ds(i * _LANES, _LANES)] = _srow(v)
                return carry
            def halve_c(i, carry):
                v = idx_c[pl.ds(i * _LANES, _LANES)]
                idx_c2[pl.ds(i * _LANES, _LANES)] = _srow(v)
                return carry
            lax.fori_loop(0, CHUNK // _LANES, halve_t, 0)
            lax.fori_loop(0, CB // _LANES, halve_c, 0)
            cp_t = pltpu.async_copy(ttab_hbm.at[idx_t2], trows, sem_t)
            cps = []
            for j in range(ctx):
                cps.append(pltpu.async_copy(
                    ctab_hbm.at[idx_c2.at[pl.ds(j * CHUNK, CHUNK)]],
                    crows.at[pl.ds(j * CHUNK, CHUNK), :], sem_c))
            cp_t.wait()
            for cp in cps:
                cp.wait()

            def g_body(g, carry):
                b_vec = g * _LANES + lane
                tv = idx_t[pl.ds(g * _LANES, _LANES)]
                tpar = (lax.shift_right_logical(tv, _NBH_SH) & 1) * dim
                crow = [b_vec * ctx + c for c in range(ctx)]
                cpar = []
                for c in range(ctx):
                    cv = plsc.load_gather(idx_c, [crow[c]])
                    cpar.append(
                        (lax.shift_right_logical(cv, _NBH_SH) & 1) * dim)
                accs = [jnp.zeros((_LANES,), jnp.float32) for _ in range(ctx)]
                for e in range(dim):
                    w = plsc.load_gather(trows, [b_vec, tpar + e])
                    for c in range(ctx):
                        x = plsc.load_gather(crows, [crow[c], cpar[c] + e])
                        accs[c] = accs[c] + w * x
                for c in range(ctx):
                    plsc.store_scatter(outv, [crow[c]], accs[c])
                return carry

            lax.fori_loop(0, CHUNK // _LANES, g_body, 0)
            pltpu.sync_copy(outv, out_hbm.at[pl.ds(off_c, CB)])

    return k


def _tc_repack(V, dim):
    NB = _NB
    grid = (V + NB - 1) // NB

    def body(in_ref, out_ref):
        x = in_ref[...]
        ident = jnp.eye(dim, dtype=jnp.float32)
        xt = lax.dot_general(x, ident, (((0,), (0,)), ((), ())),
                             preferred_element_type=jnp.float32)
        out_ref[:, 0:dim] = xt[0:NB // 2]
        out_ref[:, dim:2 * dim] = xt[NB // 2:NB]

    return pl.pallas_call(
        body,
        grid=(grid,),
        in_specs=[pl.BlockSpec((dim, NB), lambda i: (0, i))],
        out_specs=pl.BlockSpec((NB // 2, 2 * dim), lambda i: (i, 0)),
        out_shape=jax.ShapeDtypeStruct((grid * (NB // 2), 2 * dim),
                                       jnp.float32),
    )


def kernel(target, context, target_table, context_table):
    B, ctx = context.shape
    V, dim = target_table.shape
    tgt = target.reshape(B).astype(jnp.int32)
    ctxi = context.reshape(B * ctx).astype(jnp.int32)
    repack = _tc_repack(V, dim)
    ttab2 = repack(target_table.T)
    ctab2 = repack(context_table.T)
    out = _sc_word2vec(B, ctx, dim)(tgt, ctxi, ttab2, ctab2)
    return out.reshape(B, ctx)
